# Initial kernel scaffold; baseline (speedup 1.0000x reference)
import jax
import jax.numpy as jnp
from jax.experimental import pallas as pl


def kernel(x, class_embedding):
    raise NotImplementedError("write your pallas kernel here")


# speedup vs baseline: 1.1019x; 1.1019x over previous
import functools

import jax
import jax.numpy as jnp
from jax import lax
from jax.experimental import pallas as pl
from jax.experimental.pallas import tpu as pltpu
from jax.experimental.pallas import tpu_sc as plsc

_VOCAB = 1000000
_EMBED_DIM = 32
_BATCH = 16384
_FIELDS = 100

_N = _BATCH * _FIELDS
_NW = 32
_B_PER_W = _N // _NW
_CHUNK = 1024
_N_CHUNKS = _B_PER_W // _CHUNK


def _gather_body(table_hbm, idx_hbm, out_hbm, idx_v, rows_v, sem):
    c = lax.axis_index("c")
    s = lax.axis_index("s")
    wid = s * 2 + c
    base = wid * _B_PER_W

    def chunk(i, carry):
        off = base + i * _CHUNK
        pltpu.sync_copy(idx_hbm.at[pl.ds(off, _CHUNK)], idx_v)
        pltpu.async_copy(table_hbm.at[idx_v], rows_v, sem).wait()
        pltpu.sync_copy(rows_v, out_hbm.at[pl.ds(off, _CHUNK)])
        return carry

    lax.fori_loop(0, _N_CHUNKS, chunk, 0)


@jax.jit
def _gather(table, idx_flat):
    mesh = plsc.VectorSubcoreMesh(core_axis_name="c", subcore_axis_name="s")
    fn = pl.kernel(
        _gather_body,
        mesh=mesh,
        out_type=jax.ShapeDtypeStruct((_N, _EMBED_DIM), jnp.float32),
        scratch_types=[
            pltpu.VMEM((_CHUNK,), jnp.int32),
            pltpu.VMEM((_CHUNK, _EMBED_DIM), jnp.float32),
            pltpu.SemaphoreType.DMA,
        ],
        compiler_params=pltpu.CompilerParams(use_tc_tiling_on_sc=False),
    )
    return fn(table, idx_flat)


def kernel(x, class_embedding):
    idx_flat = x.reshape(-1).astype(jnp.int32)
    out = _gather(class_embedding, idx_flat)
    return out.reshape(_BATCH, _FIELDS, _EMBED_DIM)


# speedup vs baseline: 1.1126x; 1.0097x over previous
import jax
import jax.numpy as jnp
from jax import lax
from jax.experimental import pallas as pl
from jax.experimental.pallas import tpu as pltpu
from jax.experimental.pallas import tpu_sc as plsc

_VOCAB = 1000000
_EMBED_DIM = 32
_BATCH = 16384
_FIELDS = 100

_N = _BATCH * _FIELDS
_NW = 32
_B_PER_W = _N // _NW
_CHUNK = 1024
_N_CHUNKS = _B_PER_W // _CHUNK


def _gather_body(table_hbm, idx_hbm, out_hbm,
                 idx_all, rows0, rows1, gs0, gs1, ss0, ss1):
    c = lax.axis_index("c")
    s = lax.axis_index("s")
    wid = s * 2 + c
    base = wid * _B_PER_W

    pltpu.sync_copy(idx_hbm.at[pl.ds(base, _B_PER_W)], idx_all)

    rows = (rows0, rows1)
    gs = (gs0, gs1)
    ss = (ss0, ss1)

    def start_gather(ci, b):
        idx_slice = idx_all.at[pl.ds(ci * _CHUNK, _CHUNK)]
        pltpu.async_copy(table_hbm.at[idx_slice], rows[b], gs[b])

    def wait_gather(b):
        pltpu.make_async_copy(out_hbm.at[pl.ds(0, _CHUNK)], rows[b], gs[b]).wait()

    def start_store(ci, b):
        dst = out_hbm.at[pl.ds(base + ci * _CHUNK, _CHUNK)]
        pltpu.async_copy(rows[b], dst, ss[b])

    def wait_store(b):
        pltpu.make_async_copy(rows[b], out_hbm.at[pl.ds(0, _CHUNK)], ss[b]).wait()

    start_gather(0, 0)
    start_gather(1, 1)
    wait_gather(0)
    start_store(0, 0)

    def pair(g, carry):
        ci0 = 2 * g
        wait_store(0)
        start_gather(ci0, 0)
        wait_gather(1)
        start_store(ci0 - 1, 1)
        wait_store(1)
        start_gather(ci0 + 1, 1)
        wait_gather(0)
        start_store(ci0, 0)
        return carry

    lax.fori_loop(1, _N_CHUNKS // 2, pair, 0)

    wait_gather(1)
    start_store(_N_CHUNKS - 1, 1)
    wait_store(0)
    wait_store(1)


@jax.jit
def _gather(table, idx_flat):
    mesh = plsc.VectorSubcoreMesh(core_axis_name="c", subcore_axis_name="s")
    fn = pl.kernel(
        _gather_body,
        mesh=mesh,
        out_type=jax.ShapeDtypeStruct((_N, _EMBED_DIM), jnp.float32),
        scratch_types=[
            pltpu.VMEM((_B_PER_W,), jnp.int32),
            pltpu.VMEM((_CHUNK, _EMBED_DIM), jnp.float32),
            pltpu.VMEM((_CHUNK, _EMBED_DIM), jnp.float32),
            pltpu.SemaphoreType.DMA,
            pltpu.SemaphoreType.DMA,
            pltpu.SemaphoreType.DMA,
            pltpu.SemaphoreType.DMA,
        ],
        compiler_params=pltpu.CompilerParams(use_tc_tiling_on_sc=False),
    )
    return fn(table, idx_flat)


def kernel(x, class_embedding):
    idx_flat = x.reshape(-1).astype(jnp.int32)
    out = _gather(class_embedding, idx_flat)
    return out.reshape(_BATCH, _FIELDS, _EMBED_DIM)
